# baseline (speedup 1.0000x reference)
import functools

import jax
import jax.numpy as jnp
from jax import lax
from jax.experimental import pallas as pl
from jax.experimental.pallas import tpu as pltpu
from jax.experimental.pallas import tpu_sc as plsc

V_DIM = 100000
D_DIM = 64
BATCH = 4096
HIST_LEN = 50

NC = 2
NS = 16
NW = NC * NS
LANES = 16

B_PER_W = BATCH // NW
HC = 5
N_CHUNKS = HIST_LEN // HC
CHUNK = HC * B_PER_W


def _softplus_vec(v):
    a = jnp.abs(v)
    u = jnp.exp(-a)
    t = u / (2.0 + u)
    t2 = t * t
    s = 1.0 + t2 * (1.0 / 3.0 + t2 * (1.0 / 5.0 + t2 * (1.0 / 7.0 + t2 * (1.0 / 9.0))))
    return jnp.maximum(v, 0.0) + 2.0 * t * s


def _sc_body(xt_hbm, mu_t_hbm, sg_t_hbm, mu_out_hbm, sg_out_hbm,
             idx_v, g_v, s1_v, s2_v, t3_v,
             sem_idx, sem_mu, sem_sg, sem_t3, sem_s2):
    c = lax.axis_index("c")
    s = lax.axis_index("s")
    wid = s * NC + c
    b0 = wid * B_PER_W

    iota = lax.iota(jnp.int32, LANES)
    zeros = jnp.zeros((LANES,), jnp.int32)

    def load_idx(k, async_=True):
        h0 = k * HC
        cps = []
        for h in range(HC):
            cp = pltpu.make_async_copy(
                xt_hbm.at[pl.ds((h0 + h) * BATCH + b0, B_PER_W)],
                idx_v[k % 2].at[pl.ds(h * B_PER_W, B_PER_W)],
                sem_idx[k % 2],
            )
            cp.start()
            cps.append(cp)
        return cps

    def fire_gathers(k):
        p = k % 2
        cp_mu = pltpu.make_async_copy(mu_t_hbm.at[idx_v[p]], g_v[p], sem_mu[p])
        cp_mu.start()
        cp_sg = pltpu.make_async_copy(sg_t_hbm.at[idx_v[p]], s1_v[p], sem_sg[p])
        cp_sg.start()
        return cp_mu, cp_sg

    pending_idx = {0: load_idx(0)}
    pending_gather = {}
    pending_t3 = None
    pending_s2 = {}

    for cp in pending_idx.pop(0):
        cp.wait()
    pending_gather[0] = fire_gathers(0)
    pending_idx[1] = load_idx(1)

    for k in range(N_CHUNKS):
        p = k % 2
        h0 = k * HC

        if k + 1 < N_CHUNKS:
            for cp in pending_idx.pop(k + 1):
                cp.wait()
            pending_gather[k + 1] = fire_gathers(k + 1)

        cp_mu, cp_sg = pending_gather.pop(k)
        cp_sg.wait()

        if k in pending_s2:
            pending_s2.pop(k).wait()

        def sp_step(i):
            off = i * LANES
            v = _softplus_vec(s1_v[p][pl.ds(off, LANES)])
            s2_v[p][i // 8, pl.ds((i % 8) * LANES, LANES)] = v

        plsc.parallel_loop(0, CHUNK // LANES, unroll=4)(sp_step)

        cp_s2 = pltpu.make_async_copy(
            s2_v[p],
            sg_out_hbm.at[pl.ds(h0, HC), pl.ds(b0, B_PER_W)],
            sem_s2[p],
        )
        cp_s2.start()
        pending_s2[k + 2] = cp_s2

        cp_mu.wait()
        if pending_t3 is not None:
            pending_t3.wait()

        def tr_step(r):
            h = r // B_PER_W
            b = r % B_PER_W
            hvec = zeros + h
            bvec = zeros + b
            for d0 in range(0, D_DIM, LANES):
                vals = g_v[p][r, pl.ds(d0, LANES)]
                plsc.store_scatter(t3_v, [hvec, iota + d0, bvec], vals)

        plsc.parallel_loop(0, CHUNK, unroll=4)(tr_step)

        cp_t3 = pltpu.make_async_copy(
            t3_v,
            mu_out_hbm.at[pl.ds(h0, HC), :, pl.ds(b0, B_PER_W)],
            sem_t3,
        )
        cp_t3.start()
        pending_t3 = cp_t3

        if k + 2 < N_CHUNKS:
            pending_idx[k + 2] = load_idx(k + 2)

    pending_t3.wait()
    for cp in pending_s2.values():
        cp.wait()


@jax.jit
def _run(xt_flat, mu_table, sg_flat):
    mesh = plsc.VectorSubcoreMesh(core_axis_name="c", subcore_axis_name="s")
    f = pl.kernel(
        _sc_body,
        out_type=[
            jax.ShapeDtypeStruct((HIST_LEN, D_DIM, BATCH), jnp.float32),
            jax.ShapeDtypeStruct((HIST_LEN, BATCH), jnp.float32),
        ],
        mesh=mesh,
        scratch_types=[
            [pltpu.VMEM((CHUNK,), jnp.int32) for _ in range(2)],
            [pltpu.VMEM((CHUNK, D_DIM), jnp.float32) for _ in range(2)],
            [pltpu.VMEM((CHUNK,), jnp.float32) for _ in range(2)],
            [pltpu.VMEM((HC, B_PER_W), jnp.float32) for _ in range(2)],
            pltpu.VMEM((HC, D_DIM, B_PER_W), jnp.float32),
            [pltpu.SemaphoreType.DMA for _ in range(2)],
            [pltpu.SemaphoreType.DMA for _ in range(2)],
            [pltpu.SemaphoreType.DMA for _ in range(2)],
            pltpu.SemaphoreType.DMA,
            [pltpu.SemaphoreType.DMA for _ in range(2)],
        ],
        compiler_params=pltpu.CompilerParams(
            use_tc_tiling_on_sc=False, needs_layout_passes=False),
    )
    return f(xt_flat, mu_table, sg_flat)


def kernel(x, mu_table, sigma_table):
    xt_flat = x.T.reshape(BATCH * HIST_LEN)
    sg_flat = sigma_table.reshape(V_DIM)
    mu_t, sg_t = _run(xt_flat, mu_table, sg_flat)
    mu = jnp.transpose(mu_t, (2, 0, 1))
    sigma = jnp.transpose(sg_t, (1, 0)).reshape(BATCH, HIST_LEN, 1)
    return (mu, sigma)


# speedup vs baseline: 2.4437x; 2.4437x over previous
import jax
import jax.numpy as jnp
from jax import lax
from jax.experimental import pallas as pl
from jax.experimental.pallas import tpu as pltpu
from jax.experimental.pallas import tpu_sc as plsc

V_DIM = 100000
D_DIM = 64
BATCH = 4096
HIST_LEN = 50

NC = 2
NS = 16
NW = NC * NS
LANES = 16

D_PER_W = D_DIM // NW
N_IDX = BATCH * HIST_LEN
SG_PER_W = N_IDX // NW
SG_CHUNK = 640
SG_NCHUNKS = SG_PER_W // SG_CHUNK
H_VECS = BATCH // LANES


def _softplus_vec(v):
    a = jnp.abs(v)
    u = jnp.exp(-a)
    t = u / (2.0 + u)
    t2 = t * t
    s = 1.0 + t2 * (1.0 / 3.0 + t2 * (1.0 / 5.0 + t2 * (1.0 / 7.0 + t2 * (1.0 / 9.0))))
    return jnp.maximum(v, 0.0) + 2.0 * t * s


def _sc_body(xt_hbm, mu_t_hbm, sg_t_hbm, mu_out_hbm, sg_out_hbm,
             row_v, idx_v, out_v, sgi_v, sg_v,
             sem_row, sem_idx, sem_out, sem_sgi, sem_sg, sem_sgo):
    c = lax.axis_index("c")
    s = lax.axis_index("s")
    wid = s * NC + c

    sg_base = wid * SG_PER_W

    def sg_load_idx(k):
        p = k % 2
        cp = pltpu.make_async_copy(
            xt_hbm.at[pl.ds(sg_base + k * SG_CHUNK, SG_CHUNK)],
            sgi_v[p], sem_sgi[p])
        cp.start()
        return cp

    def sg_fire(k):
        p = k % 2
        cp = pltpu.make_async_copy(sg_t_hbm.at[sgi_v[p]], sg_v[p], sem_sg[p])
        cp.start()
        return cp

    pend_idx = {0: sg_load_idx(0)}
    pend_idx[0].wait()
    pend_g = {0: sg_fire(0)}
    pend_idx[1] = sg_load_idx(1)
    pend_o = {}
    for k in range(SG_NCHUNKS):
        p = k % 2
        if k + 1 < SG_NCHUNKS:
            pend_idx.pop(k + 1).wait()
            pend_g[k + 1] = sg_fire(k + 1)
        pend_g.pop(k).wait()
        if k in pend_o:
            pend_o.pop(k).wait()

        def sp_step(i):
            off = i * LANES
            sg_v[p][pl.ds(off, LANES)] = _softplus_vec(sg_v[p][pl.ds(off, LANES)])

        plsc.parallel_loop(0, SG_CHUNK // LANES, unroll=4)(sp_step)
        cp = pltpu.make_async_copy(
            sg_v[p],
            sg_out_hbm.at[pl.ds(sg_base + k * SG_CHUNK, SG_CHUNK)],
            sem_sgo[p])
        cp.start()
        pend_o[k + 2] = cp
        if k + 2 < SG_NCHUNKS:
            pend_idx[k + 2] = sg_load_idx(k + 2)
    for cp in pend_o.values():
        cp.wait()

    def mu_load_idx(h, p):
        cp = pltpu.make_async_copy(
            xt_hbm.at[pl.ds(h * BATCH, BATCH)], idx_v[p], sem_idx[p])
        cp.start()
        return cp

    for plane in range(D_PER_W):
        d = wid * D_PER_W + plane
        pltpu.make_async_copy(mu_t_hbm.at[d], row_v, sem_row).start()

        cp_i0 = mu_load_idx(0, 0)
        pltpu.make_async_copy(mu_t_hbm.at[d], row_v, sem_row).wait()

        def h_pair(i, carry):
            for par in range(2):
                h = 2 * i + par
                nxt = pltpu.make_async_copy(
                    xt_hbm.at[pl.ds((h + 1) * BATCH, BATCH)],
                    idx_v[1 - par], sem_idx[1 - par])

                @pl.when(h + 1 < HIST_LEN)
                def _start_next():
                    nxt.start()

                pltpu.make_async_copy(
                    xt_hbm.at[pl.ds(h * BATCH, BATCH)],
                    idx_v[par], sem_idx[par]).wait()

                @pl.when(i > 0)
                def _drain_prev():
                    pltpu.make_async_copy(
                        out_v[par],
                        mu_out_hbm.at[jnp.maximum(h - 2, 0), d],
                        sem_out[par]).wait()

                def g_step(j):
                    off = j * LANES
                    iv = idx_v[par][pl.ds(off, LANES)]
                    out_v[par][pl.ds(off, LANES)] = plsc.load_gather(row_v, [iv])

                plsc.parallel_loop(0, H_VECS, unroll=4)(g_step)
                pltpu.make_async_copy(
                    out_v[par], mu_out_hbm.at[h, d], sem_out[par]).start()
            return carry

        lax.fori_loop(0, HIST_LEN // 2, h_pair, None)
        for par in range(2):
            pltpu.make_async_copy(
                out_v[par],
                mu_out_hbm.at[HIST_LEN - 2 + par, d],
                sem_out[par]).wait()


@jax.jit
def _run(xt_flat, mu_tt, sg_flat):
    mesh = plsc.VectorSubcoreMesh(core_axis_name="c", subcore_axis_name="s")
    f = pl.kernel(
        _sc_body,
        out_type=[
            jax.ShapeDtypeStruct((HIST_LEN, D_DIM, BATCH), jnp.float32),
            jax.ShapeDtypeStruct((N_IDX,), jnp.float32),
        ],
        mesh=mesh,
        scratch_types=[
            pltpu.VMEM((V_DIM,), jnp.float32),
            [pltpu.VMEM((BATCH,), jnp.int32) for _ in range(2)],
            [pltpu.VMEM((BATCH,), jnp.float32) for _ in range(2)],
            [pltpu.VMEM((SG_CHUNK,), jnp.int32) for _ in range(2)],
            [pltpu.VMEM((SG_CHUNK,), jnp.float32) for _ in range(2)],
            pltpu.SemaphoreType.DMA,
            [pltpu.SemaphoreType.DMA for _ in range(2)],
            [pltpu.SemaphoreType.DMA for _ in range(2)],
            [pltpu.SemaphoreType.DMA for _ in range(2)],
            [pltpu.SemaphoreType.DMA for _ in range(2)],
            [pltpu.SemaphoreType.DMA for _ in range(2)],
        ],
        compiler_params=pltpu.CompilerParams(
            use_tc_tiling_on_sc=True, disable_bounds_checks=True,
            needs_layout_passes=False),
    )
    return f(xt_flat, mu_tt, sg_flat)


def kernel(x, mu_table, sigma_table):
    xt_flat = x.T.reshape(N_IDX)
    mu_tt = mu_table.T
    sg_flat = sigma_table.reshape(V_DIM)
    mu_t, sg_t = _run(xt_flat, mu_tt, sg_flat)
    mu = jnp.transpose(mu_t, (2, 0, 1))
    sigma = jnp.transpose(sg_t.reshape(HIST_LEN, BATCH), (1, 0)).reshape(
        BATCH, HIST_LEN, 1)
    return (mu, sigma)


# speedup vs baseline: 2.5851x; 1.0579x over previous
import jax
import jax.numpy as jnp
from jax import lax
from jax.experimental import pallas as pl
from jax.experimental.pallas import tpu as pltpu
from jax.experimental.pallas import tpu_sc as plsc

V_DIM = 100000
D_DIM = 64
BATCH = 4096
HIST_LEN = 50

NC = 2
NS = 16
NW = NC * NS
LANES = 16

D_PER_W = D_DIM // NW
N_IDX = BATCH * HIST_LEN
SG_PER_W = N_IDX // NW
SG_CHUNK = 640
SG_NCHUNKS = SG_PER_W // SG_CHUNK
H_VECS = BATCH // LANES


def _softplus_vec(v):
    a = jnp.abs(v)
    u = jnp.exp(-a)
    t = u / (2.0 + u)
    t2 = t * t
    s = 1.0 + t2 * (1.0 / 3.0 + t2 * (1.0 / 5.0 + t2 * (1.0 / 7.0 + t2 * (1.0 / 9.0))))
    return jnp.maximum(v, 0.0) + 2.0 * t * s


def _sc_body(xt_hbm, mu_t_hbm, sg_t_hbm, mu_out_hbm, sg_out_hbm,
             row_v, idx_v, out_v, sgi_v, sg_v,
             sem_row, sem_idx, sem_out, sem_sgi, sem_sg, sem_sgo):
    c = lax.axis_index("c")
    s = lax.axis_index("s")
    wid = s * NC + c

    sg_base = wid * SG_PER_W
    pltpu.make_async_copy(
        xt_hbm.at[pl.ds(sg_base, SG_PER_W)], sgi_v, sem_sgi).start()
    pltpu.make_async_copy(
        xt_hbm.at[pl.ds(sg_base, SG_PER_W)], sgi_v, sem_sgi).wait()
    pltpu.make_async_copy(sg_t_hbm.at[sgi_v], sg_v, sem_sg).start()

    def sigma_finish():
        pltpu.make_async_copy(sg_t_hbm.at[sgi_v], sg_v, sem_sg).wait()

        def sp_step(i):
            off = i * LANES
            sg_v[pl.ds(off, LANES)] = _softplus_vec(sg_v[pl.ds(off, LANES)])

        plsc.parallel_loop(0, SG_PER_W // LANES, unroll=4)(sp_step)
        pltpu.make_async_copy(
            sg_v, sg_out_hbm.at[pl.ds(sg_base, SG_PER_W)], sem_sgo).start()

    def mu_load_idx(h, p):
        cp = pltpu.make_async_copy(
            xt_hbm.at[pl.ds(h * BATCH, BATCH)], idx_v[p], sem_idx[p])
        cp.start()
        return cp

    for plane in range(D_PER_W):
        d = wid * D_PER_W + plane
        pltpu.make_async_copy(mu_t_hbm.at[d], row_v, sem_row).start()

        cp_i0 = mu_load_idx(0, 0)
        pltpu.make_async_copy(mu_t_hbm.at[d], row_v, sem_row).wait()

        def h_pair(i, carry):
            for par in range(2):
                h = 2 * i + par
                nxt = pltpu.make_async_copy(
                    xt_hbm.at[pl.ds((h + 1) * BATCH, BATCH)],
                    idx_v[1 - par], sem_idx[1 - par])

                @pl.when(h + 1 < HIST_LEN)
                def _start_next():
                    nxt.start()

                pltpu.make_async_copy(
                    xt_hbm.at[pl.ds(h * BATCH, BATCH)],
                    idx_v[par], sem_idx[par]).wait()

                @pl.when(i > 0)
                def _drain_prev():
                    pltpu.make_async_copy(
                        out_v[par],
                        mu_out_hbm.at[jnp.maximum(h - 2, 0), d],
                        sem_out[par]).wait()

                def g_step(j):
                    off = j * LANES
                    iv = idx_v[par][pl.ds(off, LANES)]
                    out_v[par][pl.ds(off, LANES)] = plsc.load_gather(row_v, [iv])

                plsc.parallel_loop(0, H_VECS, unroll=8)(g_step)
                pltpu.make_async_copy(
                    out_v[par], mu_out_hbm.at[h, d], sem_out[par]).start()
            return carry

        lax.fori_loop(0, HIST_LEN // 2, h_pair, None)
        for par in range(2):
            pltpu.make_async_copy(
                out_v[par],
                mu_out_hbm.at[HIST_LEN - 2 + par, d],
                sem_out[par]).wait()
        if plane == 0:
            sigma_finish()
    pltpu.make_async_copy(
        sg_v, sg_out_hbm.at[pl.ds(sg_base, SG_PER_W)], sem_sgo).wait()


@jax.jit
def _run(xt_flat, mu_tt, sg_flat):
    mesh = plsc.VectorSubcoreMesh(core_axis_name="c", subcore_axis_name="s")
    f = pl.kernel(
        _sc_body,
        out_type=[
            jax.ShapeDtypeStruct((HIST_LEN, D_DIM, BATCH), jnp.float32),
            jax.ShapeDtypeStruct((N_IDX,), jnp.float32),
        ],
        mesh=mesh,
        scratch_types=[
            pltpu.VMEM((V_DIM,), jnp.float32),
            [pltpu.VMEM((BATCH,), jnp.int32) for _ in range(2)],
            [pltpu.VMEM((BATCH,), jnp.float32) for _ in range(2)],
            pltpu.VMEM((SG_PER_W,), jnp.int32),
            pltpu.VMEM((SG_PER_W,), jnp.float32),
            pltpu.SemaphoreType.DMA,
            [pltpu.SemaphoreType.DMA for _ in range(2)],
            [pltpu.SemaphoreType.DMA for _ in range(2)],
            pltpu.SemaphoreType.DMA,
            pltpu.SemaphoreType.DMA,
            pltpu.SemaphoreType.DMA,
        ],
        compiler_params=pltpu.CompilerParams(
            use_tc_tiling_on_sc=True, disable_bounds_checks=True,
            needs_layout_passes=False),
    )
    return f(xt_flat, mu_tt, sg_flat)


def kernel(x, mu_table, sigma_table):
    xt_flat = x.T.reshape(N_IDX)
    mu_tt = mu_table.T
    sg_flat = sigma_table.reshape(V_DIM)
    mu_t, sg_t = _run(xt_flat, mu_tt, sg_flat)
    mu = jnp.transpose(mu_t, (2, 0, 1))
    sigma = jnp.transpose(sg_t.reshape(HIST_LEN, BATCH), (1, 0)).reshape(
        BATCH, HIST_LEN, 1)
    return (mu, sigma)


# speedup vs baseline: 3.2388x; 1.2529x over previous
import jax
import jax.numpy as jnp
from jax import lax
from jax.experimental import pallas as pl
from jax.experimental.pallas import tpu as pltpu
from jax.experimental.pallas import tpu_sc as plsc

V_DIM = 100000
D_DIM = 64
BATCH = 4096
HIST_LEN = 50

NC = 2
NS = 16
NW = NC * NS
LANES = 16

D_PER_W = D_DIM // NW
N_IDX = BATCH * HIST_LEN
SG_PER_W = N_IDX // NW
SG_SHOT = SG_PER_W // 2
H_VECS = BATCH // LANES
H_CACHED = 32


def _softplus_vec(v):
    a = jnp.abs(v)
    u = jnp.exp(-a)
    t = u / (2.0 + u)
    t2 = t * t
    s = 1.0 + t2 * (1.0 / 3.0 + t2 * (1.0 / 5.0 + t2 * (1.0 / 7.0 + t2 * (1.0 / 9.0))))
    return jnp.maximum(v, 0.0) + 2.0 * t * s


def _sc_body(xt_hbm, mu_t_hbm, sg_t_hbm, mu_out_hbm, sg_out_hbm,
             row_v, idx_v, out_v, sgi_v, sg_v, xt_sp,
             sem_row, sem_idx, sem_out, sem_sgi, sem_sg, sem_sgo, sem_xs):
    c = lax.axis_index("c")
    s = lax.axis_index("s")
    wid = s * NC + c

    pltpu.make_async_copy(mu_t_hbm.at[wid * D_PER_W], row_v, sem_row).start()
    for j in range(H_CACHED // NS):
        h = s + j * NS
        pltpu.make_async_copy(
            xt_hbm.at[pl.ds(h * BATCH, BATCH)],
            xt_sp.at[pl.ds(h * BATCH, BATCH)], sem_xs).start()

    sg_base = wid * SG_PER_W
    pltpu.make_async_copy(
        xt_hbm.at[pl.ds(sg_base, SG_SHOT)], sgi_v, sem_sgi).start()
    pltpu.make_async_copy(
        xt_hbm.at[pl.ds(sg_base, SG_SHOT)], sgi_v, sem_sgi).wait()
    pltpu.make_async_copy(sg_t_hbm.at[sgi_v], sg_v, sem_sg).start()

    for j in range(H_CACHED // NS):
        h = s + j * NS
        pltpu.make_async_copy(
            xt_hbm.at[pl.ds(h * BATCH, BATCH)],
            xt_sp.at[pl.ds(h * BATCH, BATCH)], sem_xs).wait()
    plsc.subcore_barrier()

    def sigma_finish(shot):
        base = sg_base + shot * SG_SHOT
        pltpu.make_async_copy(sg_t_hbm.at[sgi_v], sg_v, sem_sg).wait()

        def sp_step(i):
            off = i * LANES
            sg_v[pl.ds(off, LANES)] = _softplus_vec(sg_v[pl.ds(off, LANES)])

        plsc.parallel_loop(0, SG_SHOT // LANES, unroll=4)(sp_step)
        pltpu.make_async_copy(
            sg_v, sg_out_hbm.at[pl.ds(base, SG_SHOT)], sem_sgo).start()
        if shot == 0:
            pltpu.make_async_copy(
                xt_hbm.at[pl.ds(sg_base + SG_SHOT, SG_SHOT)],
                sgi_v, sem_sgi).start()
            pltpu.make_async_copy(
                xt_hbm.at[pl.ds(sg_base + SG_SHOT, SG_SHOT)],
                sgi_v, sem_sgi).wait()
            pltpu.make_async_copy(sg_t_hbm.at[sgi_v], sg_v, sem_sg).start()

    def idx_start(h, p):
        cp_sp = pltpu.make_async_copy(
            xt_sp.at[pl.ds(jnp.minimum(h, H_CACHED - 1) * BATCH, BATCH)],
            idx_v[p], sem_idx[p])
        cp_hbm = pltpu.make_async_copy(
            xt_hbm.at[pl.ds(h * BATCH, BATCH)], idx_v[p], sem_idx[p])

        @pl.when(h < H_CACHED)
        def _from_spmem():
            cp_sp.start()

        @pl.when(h >= H_CACHED)
        def _from_hbm():
            cp_hbm.start()

    for plane in range(D_PER_W):
        d = wid * D_PER_W + plane
        if plane > 0:
            pltpu.make_async_copy(mu_t_hbm.at[d], row_v, sem_row).start()

        idx_start(0, 0)
        pltpu.make_async_copy(mu_t_hbm.at[d], row_v, sem_row).wait()

        def h_pair(i, carry):
            for par in range(2):
                h = 2 * i + par
                @pl.when(h + 1 < HIST_LEN)
                def _start_next():
                    idx_start(h + 1, 1 - par)

                pltpu.make_async_copy(
                    xt_hbm.at[pl.ds(h * BATCH, BATCH)],
                    idx_v[par], sem_idx[par]).wait()

                @pl.when(i > 0)
                def _drain_prev():
                    pltpu.make_async_copy(
                        out_v[par],
                        mu_out_hbm.at[jnp.maximum(h - 2, 0), d],
                        sem_out[par]).wait()

                def g_step(j):
                    off = j * LANES
                    iv = idx_v[par][pl.ds(off, LANES)]
                    out_v[par][pl.ds(off, LANES)] = plsc.load_gather(row_v, [iv])

                plsc.parallel_loop(0, H_VECS, unroll=8)(g_step)
                pltpu.make_async_copy(
                    out_v[par], mu_out_hbm.at[h, d], sem_out[par]).start()
            return carry

        lax.fori_loop(0, HIST_LEN // 2, h_pair, None)
        for par in range(2):
            pltpu.make_async_copy(
                out_v[par],
                mu_out_hbm.at[HIST_LEN - 2 + par, d],
                sem_out[par]).wait()
        sigma_finish(plane)
    for shot in range(2):
        pltpu.make_async_copy(
            sg_v, sg_out_hbm.at[pl.ds(sg_base + shot * SG_SHOT, SG_SHOT)],
            sem_sgo).wait()


@jax.jit
def _run(xt_flat, mu_tt, sg_flat):
    mesh = plsc.VectorSubcoreMesh(core_axis_name="c", subcore_axis_name="s")
    f = pl.kernel(
        _sc_body,
        out_type=[
            jax.ShapeDtypeStruct((HIST_LEN, D_DIM, BATCH), jnp.float32),
            jax.ShapeDtypeStruct((N_IDX,), jnp.float32),
        ],
        mesh=mesh,
        scratch_types=[
            pltpu.VMEM((V_DIM,), jnp.float32),
            [pltpu.VMEM((BATCH,), jnp.int32) for _ in range(2)],
            [pltpu.VMEM((BATCH,), jnp.float32) for _ in range(2)],
            pltpu.VMEM((SG_SHOT,), jnp.int32),
            pltpu.VMEM((SG_SHOT,), jnp.float32),
            pltpu.VMEM_SHARED((H_CACHED * BATCH,), jnp.int32),
            pltpu.SemaphoreType.DMA,
            [pltpu.SemaphoreType.DMA for _ in range(2)],
            [pltpu.SemaphoreType.DMA for _ in range(2)],
            pltpu.SemaphoreType.DMA,
            pltpu.SemaphoreType.DMA,
            pltpu.SemaphoreType.DMA,
            pltpu.SemaphoreType.DMA,
        ],
        compiler_params=pltpu.CompilerParams(
            use_tc_tiling_on_sc=True, disable_bounds_checks=True,
            needs_layout_passes=False),
    )
    return f(xt_flat, mu_tt, sg_flat)


def kernel(x, mu_table, sigma_table):
    xt_flat = x.T.reshape(N_IDX)
    mu_tt = mu_table.T
    sg_flat = sigma_table.reshape(V_DIM)
    mu_t, sg_t = _run(xt_flat, mu_tt, sg_flat)
    mu = jnp.transpose(mu_t, (2, 0, 1))
    sigma = jnp.transpose(sg_t.reshape(HIST_LEN, BATCH), (1, 0)).reshape(
        BATCH, HIST_LEN, 1)
    return (mu, sigma)
